# baseline (speedup 1.0000x reference)
import functools

import jax
import jax.numpy as jnp
from jax import lax
from jax.experimental import pallas as pl
from jax.experimental.pallas import tpu as pltpu
from jax.experimental.pallas import tpu_sc as plsc

N_NODES = 10000
N_PAD = 10240
D_HID = 64
NC = 2
NS = 16
NW = NC * NS
CHUNK = 128
NBUF = 4
ROWS_PER_W = N_PAD // NS

_mesh = plsc.VectorSubcoreMesh(core_axis_name="c", subcore_axis_name="s")
_sc_params = pltpu.CompilerParams(use_tc_tiling_on_sc=False)


def _make_deg_kernel(nit):
    @functools.partial(
        pl.kernel,
        out_type=jax.ShapeDtypeStruct((NC, N_PAD), jnp.float32),
        mesh=_mesh,
        compiler_params=_sc_params,
        scratch_types=[
            pltpu.VMEM((nit, 2, CHUNK), jnp.int32),
            pltpu.VMEM((CHUNK,), jnp.float32),
            pltpu.VMEM_SHARED((N_PAD,), jnp.float32),
        ],
    )
    def deg_k(ei_hbm, zeros_hbm, ones_hbm, out_hbm, idx_v, ones_v, acc_s):
        cid = lax.axis_index("c")
        sid = lax.axis_index("s")
        wid = sid * NC + cid
        row0 = sid * ROWS_PER_W
        pltpu.sync_copy(ei_hbm.at[wid], idx_v)
        pltpu.sync_copy(ones_hbm, ones_v)
        pltpu.sync_copy(zeros_hbm, acc_s.at[pl.ds(row0, ROWS_PER_W)])
        plsc.subcore_barrier()

        def body(i, carry):
            pltpu.sync_copy(ones_v, acc_s.at[idx_v.at[i, 1]], add=True)
            return carry

        lax.fori_loop(0, nit, body, 0)
        plsc.subcore_barrier()
        pltpu.sync_copy(acc_s.at[pl.ds(row0, ROWS_PER_W)],
                        out_hbm.at[cid, pl.ds(row0, ROWS_PER_W)])

    return deg_k


def _make_mp_kernel(nit):
    @functools.partial(
        pl.kernel,
        out_type=jax.ShapeDtypeStruct((NC, N_PAD, D_HID), jnp.float32),
        mesh=_mesh,
        compiler_params=_sc_params,
        scratch_types=[
            pltpu.VMEM((nit, 2, CHUNK), jnp.int32),
            pltpu.VMEM((NBUF, CHUNK, D_HID), jnp.float32),
            pltpu.VMEM_SHARED((N_PAD, D_HID), jnp.float32),
            pltpu.SemaphoreType.DMA,
            pltpu.SemaphoreType.DMA,
            pltpu.SemaphoreType.DMA,
            pltpu.SemaphoreType.DMA,
        ],
    )
    def mp_k(g_hbm, ei_hbm, zeros_hbm, out_hbm, idx_v, rows, acc_s,
             sem0, sem1, sem2, sem3):
        sems = (sem0, sem1, sem2, sem3)
        cid = lax.axis_index("c")
        sid = lax.axis_index("s")
        wid = sid * NC + cid
        row0 = sid * ROWS_PER_W
        pltpu.sync_copy(ei_hbm.at[wid], idx_v)
        pltpu.sync_copy(zeros_hbm, acc_s.at[pl.ds(row0, ROWS_PER_W)])
        plsc.subcore_barrier()

        def issue(slot, c):
            pltpu.async_copy(g_hbm.at[idx_v.at[c, 0]], rows.at[slot],
                             sems[slot])

        def wait(slot):
            pltpu.make_async_copy(g_hbm.at[idx_v.at[0, 0]], rows.at[slot],
                                  sems[slot]).wait()

        for j in range(NBUF - 1):
            issue(j, j)

        def quad(p, carry):
            c0 = p * NBUF
            for j in range(NBUF):
                c = c0 + j
                cn = c + NBUF - 1

                @pl.when(cn < nit)
                def _():
                    issue((j + NBUF - 1) % NBUF, cn)

                wait(j)
                pltpu.sync_copy(rows.at[j], acc_s.at[idx_v.at[c, 1]],
                                add=True)
            return carry

        lax.fori_loop(0, nit // NBUF, quad, 0)
        plsc.subcore_barrier()
        pltpu.sync_copy(acc_s.at[pl.ds(row0, ROWS_PER_W)],
                        out_hbm.at[cid, pl.ds(row0, ROWS_PER_W)])

    return mp_k


def _tc1_body(degp_ref, x_ref, w1_ref, ones_ref, dinvd_ref, g1_ref):
    deg = degp_ref[0:1, :] + degp_ref[1:2, :]
    dinv = jnp.where(deg > 0.0, lax.rsqrt(deg), 0.0)
    dmat = lax.dot_general(dinv, ones_ref[...], (((0,), (0,)), ((), ())),
                           preferred_element_type=jnp.float32)
    dinvd_ref[...] = dmat
    g1_ref[...] = jnp.dot(x_ref[...], w1_ref[...],
                          preferred_element_type=jnp.float32) * dmat


def _tc2_body(sp_ref, dinvd_ref, b_ref, w2_ref, g2_ref):
    s = sp_ref[0] + sp_ref[1]
    dmat = dinvd_ref[...]
    h = jnp.maximum(s * dmat + b_ref[...], 0.0)
    g2_ref[...] = jnp.dot(h, w2_ref[...],
                          preferred_element_type=jnp.float32) * dmat


def _tc3_body(sp_ref, dinvd_ref, b_ref, wfc_ref, bfc_ref, q_ref):
    s = sp_ref[0] + sp_ref[1]
    h = jnp.maximum(s * dinvd_ref[...] + b_ref[...], 0.0)
    q_ref[...] = jnp.dot(h, wfc_ref[...],
                         preferred_element_type=jnp.float32) + bfc_ref[...]


def kernel(x, edge_index, W1, b1, W2, b2, Wfc, bfc):
    n = x.shape[0]
    e = edge_index.shape[1]
    n_cls = Wfc.shape[1]
    e_tot = e + n
    grp = NW * CHUNK * NBUF
    e_pad = -(-e_tot // grp) * grp
    nit = e_pad // (NW * CHUNK)

    src = edge_index[0].astype(jnp.int32)
    dst = edge_index[1].astype(jnp.int32)
    loop = jnp.arange(n, dtype=jnp.int32)
    padv = jnp.full((e_pad - e_tot,), n, dtype=jnp.int32)
    src_f = jnp.concatenate([src, loop, padv]).reshape(NW, nit, CHUNK)
    dst_f = jnp.concatenate([dst, loop, padv]).reshape(NW, nit, CHUNK)
    ei = jnp.stack([src_f, dst_f], axis=2)
    x_pad = jnp.pad(x, ((0, N_PAD - n), (0, 0)))

    zeros1d = jnp.zeros((ROWS_PER_W,), jnp.float32)
    ones1d = jnp.ones((CHUNK,), jnp.float32)
    zeros2d = jnp.zeros((ROWS_PER_W, D_HID), jnp.float32)
    ones_row = jnp.ones((1, D_HID), jnp.float32)

    deg_k = _make_deg_kernel(nit)
    mp_k = _make_mp_kernel(nit)

    degp = deg_k(ei, zeros1d, ones1d)

    dinv_d, g1 = pl.pallas_call(
        _tc1_body,
        out_shape=[
            jax.ShapeDtypeStruct((N_PAD, D_HID), jnp.float32),
            jax.ShapeDtypeStruct((N_PAD, D_HID), jnp.float32),
        ],
    )(degp, x_pad, W1, ones_row)

    s1p = mp_k(g1, ei, zeros2d)

    g2 = pl.pallas_call(
        _tc2_body,
        out_shape=jax.ShapeDtypeStruct((N_PAD, D_HID), jnp.float32),
    )(s1p, dinv_d, b1.reshape(1, D_HID), W2)

    s2p = mp_k(g2, ei, zeros2d)

    q = pl.pallas_call(
        _tc3_body,
        out_shape=jax.ShapeDtypeStruct((N_PAD, n_cls), jnp.float32),
    )(s2p, dinv_d, b2.reshape(1, D_HID), Wfc, bfc.reshape(1, n_cls))

    return q[:n]


# speedup vs baseline: 1.7443x; 1.7443x over previous
import functools

import jax
import jax.numpy as jnp
from jax import lax
from jax.experimental import pallas as pl
from jax.experimental.pallas import tpu as pltpu
from jax.experimental.pallas import tpu_sc as plsc

N_NODES = 10000
N_PAD = 10240
D_HID = 64
NC = 2
NS = 16
NW = NC * NS
CHUNK = 128
NBUF = 2
ROWS_PER_W = N_PAD // NS

_mesh = plsc.VectorSubcoreMesh(core_axis_name="c", subcore_axis_name="s")
_sc_params = pltpu.CompilerParams(use_tc_tiling_on_sc=False)


def _make_deg_kernel(nit):
    @functools.partial(
        pl.kernel,
        out_type=jax.ShapeDtypeStruct((NC, N_PAD), jnp.float32),
        mesh=_mesh,
        compiler_params=_sc_params,
        scratch_types=[
            pltpu.VMEM((nit, 2, CHUNK), jnp.int32),
            pltpu.VMEM((CHUNK,), jnp.float32),
            pltpu.VMEM_SHARED((N_PAD,), jnp.float32),
        ],
    )
    def deg_k(ei_hbm, zeros_hbm, ones_hbm, out_hbm, idx_v, ones_v, acc_s):
        cid = lax.axis_index("c")
        sid = lax.axis_index("s")
        wid = sid * NC + cid
        row0 = sid * ROWS_PER_W
        pltpu.sync_copy(ei_hbm.at[wid], idx_v)
        pltpu.sync_copy(ones_hbm, ones_v)
        pltpu.sync_copy(zeros_hbm, acc_s.at[pl.ds(row0, ROWS_PER_W)])
        plsc.subcore_barrier()

        def body(i, carry):
            pltpu.sync_copy(ones_v, acc_s.at[idx_v.at[i, 1]], add=True)
            return carry

        lax.fori_loop(0, nit, body, 0)
        plsc.subcore_barrier()
        pltpu.sync_copy(acc_s.at[pl.ds(row0, ROWS_PER_W)],
                        out_hbm.at[cid, pl.ds(row0, ROWS_PER_W)])

    return deg_k


def _make_mp_kernel(nit):
    @functools.partial(
        pl.kernel,
        out_type=jax.ShapeDtypeStruct((NC, N_PAD, D_HID), jnp.float32),
        mesh=_mesh,
        compiler_params=_sc_params,
        scratch_types=[
            pltpu.VMEM((nit, 2, CHUNK), jnp.int32),
            pltpu.VMEM((NBUF, CHUNK, D_HID), jnp.float32),
            pltpu.VMEM_SHARED((N_PAD, D_HID), jnp.float32),
            pltpu.SemaphoreType.DMA,
            pltpu.SemaphoreType.DMA,
        ],
    )
    def mp_k(g_hbm, ei_hbm, zeros_hbm, out_hbm, idx_v, rows, acc_s,
             sem0, sem1):
        sems = (sem0, sem1)
        cid = lax.axis_index("c")
        sid = lax.axis_index("s")
        wid = sid * NC + cid
        row0 = sid * ROWS_PER_W
        pltpu.sync_copy(ei_hbm.at[wid], idx_v)
        pltpu.sync_copy(zeros_hbm, acc_s.at[pl.ds(row0, ROWS_PER_W)])
        plsc.subcore_barrier()

        def issue(slot, c):
            pltpu.async_copy(g_hbm.at[idx_v.at[c, 0]], rows.at[slot],
                             sems[slot])

        def wait(slot):
            pltpu.make_async_copy(g_hbm.at[idx_v.at[0, 0]], rows.at[slot],
                                  sems[slot]).wait()

        for j in range(NBUF - 1):
            issue(j, j)

        def quad(p, carry):
            c0 = p * NBUF
            for j in range(NBUF):
                c = c0 + j
                cn = c + NBUF - 1

                @pl.when(cn < nit)
                def _():
                    issue((j + NBUF - 1) % NBUF, cn)

                wait(j)
                pltpu.sync_copy(rows.at[j], acc_s.at[idx_v.at[c, 1]],
                                add=True)
            return carry

        lax.fori_loop(0, nit // NBUF, quad, 0)
        plsc.subcore_barrier()
        pltpu.sync_copy(acc_s.at[pl.ds(row0, ROWS_PER_W)],
                        out_hbm.at[cid, pl.ds(row0, ROWS_PER_W)])

    return mp_k


def _tc1_body(degp_ref, x_ref, w1_ref, ones_ref, dinvd_ref, g1_ref):
    deg = degp_ref[0:1, :] + degp_ref[1:2, :]
    dinv = jnp.where(deg > 0.0, lax.rsqrt(deg), 0.0)
    dmat = lax.dot_general(dinv, ones_ref[...], (((0,), (0,)), ((), ())),
                           preferred_element_type=jnp.float32)
    dinvd_ref[...] = dmat
    g1_ref[...] = jnp.dot(x_ref[...], w1_ref[...],
                          preferred_element_type=jnp.float32) * dmat


def _tc2_body(sp_ref, dinvd_ref, b_ref, w2_ref, g2_ref):
    s = sp_ref[0] + sp_ref[1]
    dmat = dinvd_ref[...]
    h = jnp.maximum(s * dmat + b_ref[...], 0.0)
    g2_ref[...] = jnp.dot(h, w2_ref[...],
                          preferred_element_type=jnp.float32) * dmat


def _tc3_body(sp_ref, dinvd_ref, b_ref, wfc_ref, bfc_ref, q_ref):
    s = sp_ref[0] + sp_ref[1]
    h = jnp.maximum(s * dinvd_ref[...] + b_ref[...], 0.0)
    q_ref[...] = jnp.dot(h, wfc_ref[...],
                         preferred_element_type=jnp.float32) + bfc_ref[...]


def kernel(x, edge_index, W1, b1, W2, b2, Wfc, bfc):
    n = x.shape[0]
    e = edge_index.shape[1]
    n_cls = Wfc.shape[1]
    e_tot = e + n
    grp = NW * CHUNK * NBUF
    e_pad = -(-e_tot // grp) * grp
    nit = e_pad // (NW * CHUNK)

    src = edge_index[0].astype(jnp.int32)
    dst = edge_index[1].astype(jnp.int32)
    loop = jnp.arange(n, dtype=jnp.int32)
    padv = jnp.full((e_pad - e_tot,), n, dtype=jnp.int32)
    src_f = jnp.concatenate([src, loop, padv]).reshape(NW, nit, CHUNK)
    dst_f = jnp.concatenate([dst, loop, padv]).reshape(NW, nit, CHUNK)
    ei = jnp.stack([src_f, dst_f], axis=2)
    x_pad = jnp.pad(x, ((0, N_PAD - n), (0, 0)))

    zeros1d = jnp.zeros((ROWS_PER_W,), jnp.float32)
    ones1d = jnp.ones((CHUNK,), jnp.float32)
    zeros2d = jnp.zeros((ROWS_PER_W, D_HID), jnp.float32)
    ones_row = jnp.ones((1, D_HID), jnp.float32)

    deg_k = _make_deg_kernel(nit)
    mp_k = _make_mp_kernel(nit)

    degp = deg_k(ei, zeros1d, ones1d)

    dinv_d, g1 = pl.pallas_call(
        _tc1_body,
        out_shape=[
            jax.ShapeDtypeStruct((N_PAD, D_HID), jnp.float32),
            jax.ShapeDtypeStruct((N_PAD, D_HID), jnp.float32),
        ],
    )(degp, x_pad, W1, ones_row)

    s1p = mp_k(g1, ei, zeros2d)

    g2 = pl.pallas_call(
        _tc2_body,
        out_shape=jax.ShapeDtypeStruct((N_PAD, D_HID), jnp.float32),
    )(s1p, dinv_d, b1.reshape(1, D_HID), W2)

    s2p = mp_k(g2, ei, zeros2d)

    q = pl.pallas_call(
        _tc3_body,
        out_shape=jax.ShapeDtypeStruct((N_PAD, n_cls), jnp.float32),
    )(s2p, dinv_d, b2.reshape(1, D_HID), Wfc, bfc.reshape(1, n_cls))

    return q[:n]


# speedup vs baseline: 2.7599x; 1.5822x over previous
import functools

import jax
import jax.numpy as jnp
from jax import lax
from jax.experimental import pallas as pl
from jax.experimental.pallas import tpu as pltpu
from jax.experimental.pallas import tpu_sc as plsc

N_NODES = 10000
N_PAD = 10240
D_HID = 64
NC = 2
NS = 16
NW = NC * NS
CHUNK = 128
NBUF = 2
ROWS_PER_W = N_PAD // NS

_mesh = plsc.VectorSubcoreMesh(core_axis_name="c", subcore_axis_name="s")
_sc_params = pltpu.CompilerParams(use_tc_tiling_on_sc=False)


def _make_deg_kernel(nit):
    @functools.partial(
        pl.kernel,
        out_type=jax.ShapeDtypeStruct((NC, N_PAD), jnp.float32),
        mesh=_mesh,
        compiler_params=_sc_params,
        scratch_types=[
            pltpu.VMEM((nit, 2, CHUNK), jnp.int32),
            pltpu.VMEM((CHUNK,), jnp.float32),
            pltpu.VMEM_SHARED((N_PAD,), jnp.float32),
        ],
    )
    def deg_k(ei_hbm, zeros_hbm, ones_hbm, out_hbm, idx_v, ones_v, acc_s):
        cid = lax.axis_index("c")
        sid = lax.axis_index("s")
        wid = sid * NC + cid
        row0 = sid * ROWS_PER_W
        pltpu.sync_copy(ei_hbm.at[wid], idx_v)
        pltpu.sync_copy(ones_hbm, ones_v)
        pltpu.sync_copy(zeros_hbm, acc_s.at[pl.ds(row0, ROWS_PER_W)])
        plsc.subcore_barrier()

        def body(i, carry):
            pltpu.sync_copy(ones_v, acc_s.at[idx_v.at[i, 1]], add=True)
            return carry

        lax.fori_loop(0, nit, body, 0)
        plsc.subcore_barrier()
        pltpu.sync_copy(acc_s.at[pl.ds(row0, ROWS_PER_W)],
                        out_hbm.at[cid, pl.ds(row0, ROWS_PER_W)])

    return deg_k


def _make_mp_kernel(nit):
    @functools.partial(
        pl.kernel,
        out_type=jax.ShapeDtypeStruct((NC, N_PAD, D_HID), jnp.float32),
        mesh=_mesh,
        compiler_params=_sc_params,
        scratch_types=[
            pltpu.VMEM((nit, 2, CHUNK), jnp.int32),
            pltpu.VMEM((NBUF, CHUNK, D_HID), jnp.float32),
            pltpu.VMEM_SHARED((N_PAD, D_HID), jnp.float32),
            pltpu.VMEM_SHARED((N_PAD, D_HID), jnp.float32),
            pltpu.SemaphoreType.DMA,
            pltpu.SemaphoreType.DMA,
        ],
    )
    def mp_k(g_hbm, ei_hbm, zeros_hbm, out_hbm, idx_v, rows, g_s, acc_s,
             sem0, sem1):
        sems = (sem0, sem1)
        cid = lax.axis_index("c")
        sid = lax.axis_index("s")
        wid = sid * NC + cid
        row0 = sid * ROWS_PER_W
        pltpu.sync_copy(ei_hbm.at[wid], idx_v)
        pltpu.sync_copy(g_hbm.at[pl.ds(row0, ROWS_PER_W)],
                        g_s.at[pl.ds(row0, ROWS_PER_W)])
        pltpu.sync_copy(zeros_hbm, acc_s.at[pl.ds(row0, ROWS_PER_W)])
        plsc.subcore_barrier()

        def issue(slot, c):
            pltpu.async_copy(g_s.at[idx_v.at[c, 0]], rows.at[slot],
                             sems[slot])

        def wait(slot):
            pltpu.make_async_copy(g_s.at[idx_v.at[0, 0]], rows.at[slot],
                                  sems[slot]).wait()

        for j in range(NBUF - 1):
            issue(j, j)

        def quad(p, carry):
            c0 = p * NBUF
            for j in range(NBUF):
                c = c0 + j
                cn = c + NBUF - 1

                @pl.when(cn < nit)
                def _():
                    issue((j + NBUF - 1) % NBUF, cn)

                wait(j)
                pltpu.sync_copy(rows.at[j], acc_s.at[idx_v.at[c, 1]],
                                add=True)
            return carry

        lax.fori_loop(0, nit // NBUF, quad, 0)
        plsc.subcore_barrier()
        pltpu.sync_copy(acc_s.at[pl.ds(row0, ROWS_PER_W)],
                        out_hbm.at[cid, pl.ds(row0, ROWS_PER_W)])

    return mp_k


def _tc1_body(degp_ref, x_ref, w1_ref, ones_ref, dinvd_ref, g1_ref):
    deg = degp_ref[0:1, :] + degp_ref[1:2, :]
    dinv = jnp.where(deg > 0.0, lax.rsqrt(deg), 0.0)
    dmat = lax.dot_general(dinv, ones_ref[...], (((0,), (0,)), ((), ())),
                           preferred_element_type=jnp.float32)
    dinvd_ref[...] = dmat
    g1_ref[...] = jnp.dot(x_ref[...], w1_ref[...],
                          preferred_element_type=jnp.float32) * dmat


def _tc2_body(sp_ref, dinvd_ref, b_ref, w2_ref, g2_ref):
    s = sp_ref[0] + sp_ref[1]
    dmat = dinvd_ref[...]
    h = jnp.maximum(s * dmat + b_ref[...], 0.0)
    g2_ref[...] = jnp.dot(h, w2_ref[...],
                          preferred_element_type=jnp.float32) * dmat


def _tc3_body(sp_ref, dinvd_ref, b_ref, wfc_ref, bfc_ref, q_ref):
    s = sp_ref[0] + sp_ref[1]
    h = jnp.maximum(s * dinvd_ref[...] + b_ref[...], 0.0)
    q_ref[...] = jnp.dot(h, wfc_ref[...],
                         preferred_element_type=jnp.float32) + bfc_ref[...]


def kernel(x, edge_index, W1, b1, W2, b2, Wfc, bfc):
    n = x.shape[0]
    e = edge_index.shape[1]
    n_cls = Wfc.shape[1]
    e_tot = e + n
    grp = NW * CHUNK * NBUF
    e_pad = -(-e_tot // grp) * grp
    nit = e_pad // (NW * CHUNK)

    src = edge_index[0].astype(jnp.int32)
    dst = edge_index[1].astype(jnp.int32)
    loop = jnp.arange(n, dtype=jnp.int32)
    padv = jnp.full((e_pad - e_tot,), n, dtype=jnp.int32)
    src_f = jnp.concatenate([src, loop, padv]).reshape(NW, nit, CHUNK)
    dst_f = jnp.concatenate([dst, loop, padv]).reshape(NW, nit, CHUNK)
    ei = jnp.stack([src_f, dst_f], axis=2)
    x_pad = jnp.pad(x, ((0, N_PAD - n), (0, 0)))

    zeros1d = jnp.zeros((ROWS_PER_W,), jnp.float32)
    ones1d = jnp.ones((CHUNK,), jnp.float32)
    zeros2d = jnp.zeros((ROWS_PER_W, D_HID), jnp.float32)
    ones_row = jnp.ones((1, D_HID), jnp.float32)

    deg_k = _make_deg_kernel(nit)
    mp_k = _make_mp_kernel(nit)

    degp = deg_k(ei, zeros1d, ones1d)

    dinv_d, g1 = pl.pallas_call(
        _tc1_body,
        out_shape=[
            jax.ShapeDtypeStruct((N_PAD, D_HID), jnp.float32),
            jax.ShapeDtypeStruct((N_PAD, D_HID), jnp.float32),
        ],
    )(degp, x_pad, W1, ones_row)

    s1p = mp_k(g1, ei, zeros2d)

    g2 = pl.pallas_call(
        _tc2_body,
        out_shape=jax.ShapeDtypeStruct((N_PAD, D_HID), jnp.float32),
    )(s1p, dinv_d, b1.reshape(1, D_HID), W2)

    s2p = mp_k(g2, ei, zeros2d)

    q = pl.pallas_call(
        _tc3_body,
        out_shape=jax.ShapeDtypeStruct((N_PAD, n_cls), jnp.float32),
    )(s2p, dinv_d, b2.reshape(1, D_HID), Wfc, bfc.reshape(1, n_cls))

    return q[:n]


# speedup vs baseline: 2.8881x; 1.0464x over previous
import functools

import jax
import jax.numpy as jnp
from jax import lax
from jax.experimental import pallas as pl
from jax.experimental.pallas import tpu as pltpu
from jax.experimental.pallas import tpu_sc as plsc

N_NODES = 10000
N_PAD = 10240
D_HID = 64
NC = 2
NS = 16
NW = NC * NS
CHUNK = 128
NBUF = 3
ROWS_PER_W = N_PAD // NS

_mesh = plsc.VectorSubcoreMesh(core_axis_name="c", subcore_axis_name="s")
_sc_params = pltpu.CompilerParams(use_tc_tiling_on_sc=False)


def _make_deg_kernel(nit):
    @functools.partial(
        pl.kernel,
        out_type=jax.ShapeDtypeStruct((NC, N_PAD), jnp.float32),
        mesh=_mesh,
        compiler_params=_sc_params,
        scratch_types=[
            pltpu.VMEM((nit, CHUNK), jnp.int32),
            pltpu.VMEM((CHUNK,), jnp.float32),
            pltpu.VMEM_SHARED((N_PAD,), jnp.float32),
        ],
    )
    def deg_k(dst_hbm, zeros_hbm, ones_hbm, out_hbm, idx_v, ones_v, acc_s):
        cid = lax.axis_index("c")
        sid = lax.axis_index("s")
        wid = sid * NC + cid
        row0 = sid * ROWS_PER_W
        pltpu.sync_copy(dst_hbm.at[wid], idx_v)
        pltpu.sync_copy(ones_hbm, ones_v)
        pltpu.sync_copy(zeros_hbm, acc_s.at[pl.ds(row0, ROWS_PER_W)])
        plsc.subcore_barrier()

        def body(i, carry):
            pltpu.sync_copy(ones_v, acc_s.at[idx_v.at[i]], add=True)
            return carry

        lax.fori_loop(0, nit, body, 0)
        plsc.subcore_barrier()
        pltpu.sync_copy(acc_s.at[pl.ds(row0, ROWS_PER_W)],
                        out_hbm.at[cid, pl.ds(row0, ROWS_PER_W)])

    return deg_k


def _make_mp_kernel(nit):
    @functools.partial(
        pl.kernel,
        out_type=jax.ShapeDtypeStruct((NC, N_PAD, D_HID), jnp.float32),
        mesh=_mesh,
        compiler_params=_sc_params,
        scratch_types=[
            pltpu.VMEM((nit, CHUNK), jnp.int32),
            pltpu.VMEM((nit, CHUNK), jnp.int32),
            pltpu.VMEM((NBUF, CHUNK, D_HID), jnp.float32),
            pltpu.VMEM_SHARED((N_PAD, D_HID), jnp.float32),
            pltpu.VMEM_SHARED((N_PAD, D_HID), jnp.float32),
            pltpu.SemaphoreType.DMA,
            pltpu.SemaphoreType.DMA,
            pltpu.SemaphoreType.DMA,
        ],
    )
    def mp_k(g_hbm, src_hbm, dst_hbm, zeros_hbm, out_hbm,
             sidx_v, didx_v, rows, g_s, acc_s, sem0, sem1, sem2):
        sems = (sem0, sem1, sem2)
        cid = lax.axis_index("c")
        sid = lax.axis_index("s")
        wid = sid * NC + cid
        row0 = sid * ROWS_PER_W
        pltpu.sync_copy(src_hbm.at[wid], sidx_v)
        pltpu.sync_copy(dst_hbm.at[wid], didx_v)
        pltpu.sync_copy(g_hbm.at[pl.ds(row0, ROWS_PER_W)],
                        g_s.at[pl.ds(row0, ROWS_PER_W)])
        pltpu.sync_copy(zeros_hbm, acc_s.at[pl.ds(row0, ROWS_PER_W)])
        plsc.subcore_barrier()

        def issue(slot, c):
            pltpu.async_copy(g_s.at[sidx_v.at[c]], rows.at[slot],
                             sems[slot])

        def wait(slot):
            pltpu.make_async_copy(g_s.at[sidx_v.at[0]], rows.at[slot],
                                  sems[slot]).wait()

        for j in range(NBUF - 1):
            issue(j, j)

        def step(p, carry):
            c0 = p * NBUF
            for j in range(NBUF):
                c = c0 + j
                cn = c + NBUF - 1

                @pl.when(cn < nit)
                def _():
                    issue((j + NBUF - 1) % NBUF, cn)

                wait(j)
                pltpu.sync_copy(rows.at[j], acc_s.at[didx_v.at[c]],
                                add=True)
            return carry

        lax.fori_loop(0, nit // NBUF, step, 0)
        plsc.subcore_barrier()
        pltpu.sync_copy(acc_s.at[pl.ds(row0, ROWS_PER_W)],
                        out_hbm.at[cid, pl.ds(row0, ROWS_PER_W)])

    return mp_k


def _dmat(degp_ref, ones_ref):
    deg = degp_ref[0:1, :] + degp_ref[1:2, :]
    dinv = jnp.where(deg > 0.0, lax.rsqrt(deg), 0.0)
    return lax.dot_general(dinv, ones_ref[...], (((0,), (0,)), ((), ())),
                           preferred_element_type=jnp.float32)


def _tc1_body(degp_ref, x_ref, w1_ref, ones_ref, g1_ref):
    n = x_ref.shape[0]
    dmat = _dmat(degp_ref, ones_ref)
    g1_ref[0:n] = jnp.dot(x_ref[...], w1_ref[...],
                          preferred_element_type=jnp.float32) * dmat[0:n]
    g1_ref[n:] = jnp.zeros((N_PAD - n, D_HID), jnp.float32)


def _tc2_body(sp_ref, degp_ref, ones_ref, b_ref, w2_ref, g2_ref):
    dmat = _dmat(degp_ref, ones_ref)
    s = sp_ref[0] + sp_ref[1]
    h = jnp.maximum(s * dmat + b_ref[...], 0.0)
    g2_ref[...] = jnp.dot(h, w2_ref[...],
                          preferred_element_type=jnp.float32) * dmat


def _tc3_body(sp_ref, degp_ref, ones_ref, b_ref, wfc_ref, bfc_ref, q_ref):
    dmat = _dmat(degp_ref, ones_ref)
    s = sp_ref[0] + sp_ref[1]
    h = jnp.maximum(s * dmat + b_ref[...], 0.0)
    q_ref[...] = jnp.dot(h, wfc_ref[...],
                         preferred_element_type=jnp.float32) + bfc_ref[...]


def kernel(x, edge_index, W1, b1, W2, b2, Wfc, bfc):
    n = x.shape[0]
    e = edge_index.shape[1]
    n_cls = Wfc.shape[1]
    e_tot = e + n
    grp = NW * CHUNK * NBUF
    e_pad = -(-e_tot // grp) * grp
    nit = e_pad // (NW * CHUNK)

    src = edge_index[0].astype(jnp.int32)
    dst = edge_index[1].astype(jnp.int32)
    loop = jnp.arange(n, dtype=jnp.int32)
    padv = jnp.full((e_pad - e_tot,), n, dtype=jnp.int32)
    src3 = jnp.concatenate([src, loop, padv]).reshape(NW, nit, CHUNK)
    dst3 = jnp.concatenate([dst, loop, padv]).reshape(NW, nit, CHUNK)

    zeros1d = jnp.zeros((ROWS_PER_W,), jnp.float32)
    ones1d = jnp.ones((CHUNK,), jnp.float32)
    zeros2d = jnp.zeros((ROWS_PER_W, D_HID), jnp.float32)
    ones_row = jnp.ones((1, D_HID), jnp.float32)

    deg_k = _make_deg_kernel(nit)
    mp_k = _make_mp_kernel(nit)

    degp = deg_k(dst3, zeros1d, ones1d)

    g1 = pl.pallas_call(
        _tc1_body,
        out_shape=jax.ShapeDtypeStruct((N_PAD, D_HID), jnp.float32),
    )(degp, x, W1, ones_row)

    s1p = mp_k(g1, src3, dst3, zeros2d)

    g2 = pl.pallas_call(
        _tc2_body,
        out_shape=jax.ShapeDtypeStruct((N_PAD, D_HID), jnp.float32),
    )(s1p, degp, ones_row, b1.reshape(1, D_HID), W2)

    s2p = mp_k(g2, src3, dst3, zeros2d)

    q = pl.pallas_call(
        _tc3_body,
        out_shape=jax.ShapeDtypeStruct((N_PAD, n_cls), jnp.float32),
    )(s2p, degp, ones_row, b2.reshape(1, D_HID), Wfc, bfc.reshape(1, n_cls))

    return q[:n]


# speedup vs baseline: 3.0433x; 1.0538x over previous
import functools

import jax
import jax.numpy as jnp
from jax import lax
from jax.experimental import pallas as pl
from jax.experimental.pallas import tpu as pltpu
from jax.experimental.pallas import tpu_sc as plsc

N_NODES = 10000
N_PAD = 10240
D_HID = 64
NC = 2
NS = 16
NW = NC * NS
CHUNK = 128
NBUF = 3
ROWS_PER_W = N_PAD // NS

_mesh = plsc.VectorSubcoreMesh(core_axis_name="c", subcore_axis_name="s")
_sc_params = pltpu.CompilerParams(use_tc_tiling_on_sc=False)


def _make_deg_kernel(nit, w_split):
    @functools.partial(
        pl.kernel,
        out_type=jax.ShapeDtypeStruct((NC, N_PAD), jnp.float32),
        mesh=_mesh,
        compiler_params=_sc_params,
        scratch_types=[
            pltpu.VMEM((nit, 2, CHUNK), jnp.int32),
            pltpu.VMEM((CHUNK,), jnp.float32),
            pltpu.VMEM_SHARED((N_PAD,), jnp.float32),
        ],
    )
    def deg_k(ei_hbm, tail_hbm, zeros_hbm, ones_hbm, out_hbm,
              idx_v, ones_v, acc_s):
        cid = lax.axis_index("c")
        sid = lax.axis_index("s")
        wid = sid * NC + cid
        row0 = sid * ROWS_PER_W

        @pl.when(wid < w_split)
        def _():
            pltpu.sync_copy(ei_hbm.at[pl.ds(wid * nit, nit)], idx_v)

        @pl.when(wid >= w_split)
        def _():
            pltpu.sync_copy(tail_hbm.at[pl.ds((wid - w_split) * nit, nit)],
                            idx_v)
        pltpu.sync_copy(ones_hbm, ones_v)
        pltpu.sync_copy(zeros_hbm, acc_s.at[pl.ds(row0, ROWS_PER_W)])
        plsc.subcore_barrier()

        def body(i, carry):
            pltpu.sync_copy(ones_v, acc_s.at[idx_v.at[i, 1]], add=True)
            return carry

        lax.fori_loop(0, nit, body, 0)
        plsc.subcore_barrier()
        pltpu.sync_copy(acc_s.at[pl.ds(row0, ROWS_PER_W)],
                        out_hbm.at[cid, pl.ds(row0, ROWS_PER_W)])

    return deg_k


def _make_mp_kernel(nit, w_split):
    @functools.partial(
        pl.kernel,
        out_type=jax.ShapeDtypeStruct((NC, N_PAD, D_HID), jnp.float32),
        mesh=_mesh,
        compiler_params=_sc_params,
        scratch_types=[
            pltpu.VMEM((nit, 2, CHUNK), jnp.int32),
            pltpu.VMEM((NBUF, CHUNK, D_HID), jnp.float32),
            pltpu.VMEM_SHARED((N_PAD, D_HID), jnp.float32),
            pltpu.VMEM_SHARED((N_PAD, D_HID), jnp.float32),
            pltpu.SemaphoreType.DMA,
            pltpu.SemaphoreType.DMA,
            pltpu.SemaphoreType.DMA,
        ],
    )
    def mp_k(g_hbm, ei_hbm, tail_hbm, zeros_hbm, out_hbm,
             idx_v, rows, g_s, acc_s, sem0, sem1, sem2):
        sems = (sem0, sem1, sem2)
        cid = lax.axis_index("c")
        sid = lax.axis_index("s")
        wid = sid * NC + cid
        row0 = sid * ROWS_PER_W

        @pl.when(wid < w_split)
        def _():
            pltpu.sync_copy(ei_hbm.at[pl.ds(wid * nit, nit)], idx_v)

        @pl.when(wid >= w_split)
        def _():
            pltpu.sync_copy(tail_hbm.at[pl.ds((wid - w_split) * nit, nit)],
                            idx_v)
        pltpu.sync_copy(g_hbm.at[pl.ds(row0, ROWS_PER_W)],
                        g_s.at[pl.ds(row0, ROWS_PER_W)])
        pltpu.sync_copy(zeros_hbm, acc_s.at[pl.ds(row0, ROWS_PER_W)])
        plsc.subcore_barrier()

        def issue(slot, c):
            pltpu.async_copy(g_s.at[idx_v.at[c, 0]], rows.at[slot],
                             sems[slot])

        def wait(slot):
            pltpu.make_async_copy(g_s.at[idx_v.at[0, 0]], rows.at[slot],
                                  sems[slot]).wait()

        for j in range(NBUF - 1):
            issue(j, j)

        def step(p, carry):
            c0 = p * NBUF
            for j in range(NBUF):
                c = c0 + j
                cn = c + NBUF - 1

                @pl.when(cn < nit)
                def _():
                    issue((j + NBUF - 1) % NBUF, cn)

                wait(j)
                pltpu.sync_copy(rows.at[j], acc_s.at[idx_v.at[c, 1]],
                                add=True)
            return carry

        lax.fori_loop(0, nit // NBUF, step, 0)
        plsc.subcore_barrier()
        pltpu.sync_copy(acc_s.at[pl.ds(row0, ROWS_PER_W)],
                        out_hbm.at[cid, pl.ds(row0, ROWS_PER_W)])

    return mp_k


def _dmat(degp_ref, ones_ref):
    deg = degp_ref[0:1, :] + degp_ref[1:2, :]
    dinv = jnp.where(deg > 0.0, lax.rsqrt(deg), 0.0)
    return lax.dot_general(dinv, ones_ref[...], (((0,), (0,)), ((), ())),
                           preferred_element_type=jnp.float32)


def _tc1_body(degp_ref, x_ref, w1_ref, ones_ref, g1_ref):
    n = x_ref.shape[0]
    dmat = _dmat(degp_ref, ones_ref)
    g1_ref[0:n] = jnp.dot(x_ref[...], w1_ref[...],
                          preferred_element_type=jnp.float32) * dmat[0:n]
    g1_ref[n:] = jnp.zeros((N_PAD - n, D_HID), jnp.float32)


def _tc2_body(sp_ref, degp_ref, ones_ref, b_ref, w2_ref, g2_ref):
    dmat = _dmat(degp_ref, ones_ref)
    s = sp_ref[0] + sp_ref[1]
    h = jnp.maximum(s * dmat + b_ref[...], 0.0)
    g2_ref[...] = jnp.dot(h, w2_ref[...],
                          preferred_element_type=jnp.float32) * dmat


def _tc3_body(sp_ref, degp_ref, ones_ref, b_ref, wfc_ref, bfc_ref, q_ref):
    dmat = _dmat(degp_ref, ones_ref)
    s = sp_ref[0] + sp_ref[1]
    h = jnp.maximum(s * dmat + b_ref[...], 0.0)
    q_ref[...] = jnp.dot(h, wfc_ref[...],
                         preferred_element_type=jnp.float32) + bfc_ref[...]


def kernel(x, edge_index, W1, b1, W2, b2, Wfc, bfc):
    n = x.shape[0]
    e = edge_index.shape[1]
    n_cls = Wfc.shape[1]
    e_tot = e + n
    grp = NW * CHUNK * NBUF
    e_pad = -(-e_tot // grp) * grp
    nit = e_pad // (NW * CHUNK)

    epw = nit * CHUNK
    w_split = e // epw
    ei32 = edge_index.astype(jnp.int32)
    ei_packed = jnp.stack(
        [ei32[0].reshape(-1, CHUNK), ei32[1].reshape(-1, CHUNK)],
        axis=1)
    loop = jnp.arange(n, dtype=jnp.int32)
    padv = jnp.full((e_pad - e_tot,), n, dtype=jnp.int32)
    tail_src = jnp.concatenate([ei32[0, w_split * epw:], loop, padv])
    tail_dst = jnp.concatenate([ei32[1, w_split * epw:], loop, padv])
    tail_packed = jnp.stack(
        [tail_src.reshape(-1, CHUNK), tail_dst.reshape(-1, CHUNK)],
        axis=1)

    zeros1d = jnp.zeros((ROWS_PER_W,), jnp.float32)
    ones1d = jnp.ones((CHUNK,), jnp.float32)
    zeros2d = jnp.zeros((ROWS_PER_W, D_HID), jnp.float32)
    ones_row = jnp.ones((1, D_HID), jnp.float32)

    deg_k = _make_deg_kernel(nit, w_split)
    mp_k = _make_mp_kernel(nit, w_split)

    degp = deg_k(ei_packed, tail_packed, zeros1d, ones1d)

    g1 = pl.pallas_call(
        _tc1_body,
        out_shape=jax.ShapeDtypeStruct((N_PAD, D_HID), jnp.float32),
    )(degp, x, W1, ones_row)

    s1p = mp_k(g1, ei_packed, tail_packed, zeros2d)

    g2 = pl.pallas_call(
        _tc2_body,
        out_shape=jax.ShapeDtypeStruct((N_PAD, D_HID), jnp.float32),
    )(s1p, degp, ones_row, b1.reshape(1, D_HID), W2)

    s2p = mp_k(g2, ei_packed, tail_packed, zeros2d)

    q = pl.pallas_call(
        _tc3_body,
        out_shape=jax.ShapeDtypeStruct((N_PAD, n_cls), jnp.float32),
    )(s2p, degp, ones_row, b2.reshape(1, D_HID), Wfc, bfc.reshape(1, n_cls))

    return q[:n]


# speedup vs baseline: 3.0671x; 1.0078x over previous
import functools

import jax
import jax.numpy as jnp
from jax import lax
from jax.experimental import pallas as pl
from jax.experimental.pallas import tpu as pltpu
from jax.experimental.pallas import tpu_sc as plsc

N_NODES = 10000
N_PAD = 10240
D_HID = 64
NC = 2
NS = 16
NW = NC * NS
CHUNK = 128
NBUF = 3
ROWS_PER_W = N_PAD // NS

_mesh = plsc.VectorSubcoreMesh(core_axis_name="c", subcore_axis_name="s")
_sc_params = pltpu.CompilerParams(use_tc_tiling_on_sc=False)


def _make_deg_kernel(nit, w_split):
    @functools.partial(
        pl.kernel,
        out_type=jax.ShapeDtypeStruct((NC, N_PAD), jnp.float32),
        mesh=_mesh,
        compiler_params=_sc_params,
        scratch_types=[
            pltpu.VMEM((nit, 2, CHUNK), jnp.int32),
            pltpu.VMEM((CHUNK,), jnp.float32),
            pltpu.VMEM_SHARED((N_PAD,), jnp.float32),
        ],
    )
    def deg_k(ei_hbm, tail_hbm, zeros_hbm, ones_hbm, out_hbm,
              idx_v, ones_v, acc_s):
        cid = lax.axis_index("c")
        sid = lax.axis_index("s")
        wid = sid * NC + cid
        row0 = sid * ROWS_PER_W

        @pl.when(wid < w_split)
        def _():
            pltpu.sync_copy(ei_hbm.at[pl.ds(wid * nit, nit)], idx_v)

        @pl.when(wid >= w_split)
        def _():
            pltpu.sync_copy(tail_hbm.at[pl.ds((wid - w_split) * nit, nit)],
                            idx_v)
        pltpu.sync_copy(ones_hbm, ones_v)
        pltpu.sync_copy(zeros_hbm, acc_s.at[pl.ds(row0, ROWS_PER_W)])
        plsc.subcore_barrier()

        def body(i, carry):
            pltpu.sync_copy(ones_v, acc_s.at[idx_v.at[i, 1]], add=True)
            return carry

        lax.fori_loop(0, nit, body, 0)
        plsc.subcore_barrier()
        pltpu.sync_copy(acc_s.at[pl.ds(row0, ROWS_PER_W)],
                        out_hbm.at[cid, pl.ds(row0, ROWS_PER_W)])

    return deg_k


def _make_mp_kernel(nit, w_split):
    @functools.partial(
        pl.kernel,
        out_type=jax.ShapeDtypeStruct((NC, N_PAD, D_HID), jnp.float32),
        mesh=_mesh,
        compiler_params=_sc_params,
        scratch_types=[
            pltpu.VMEM((nit, 2, CHUNK), jnp.int32),
            pltpu.VMEM((NBUF, CHUNK, D_HID), jnp.float32),
            pltpu.VMEM_SHARED((N_PAD, D_HID), jnp.float32),
            pltpu.VMEM_SHARED((N_PAD, D_HID), jnp.float32),
            pltpu.SemaphoreType.DMA,
            pltpu.SemaphoreType.DMA,
            pltpu.SemaphoreType.DMA,
        ],
    )
    def mp_k(g_hbm, ei_hbm, tail_hbm, zeros_hbm, out_hbm,
             idx_v, rows, g_s, acc_s, sem0, sem1, sem2):
        sems = (sem0, sem1, sem2)
        cid = lax.axis_index("c")
        sid = lax.axis_index("s")
        wid = sid * NC + cid
        row0 = sid * ROWS_PER_W

        cg = pltpu.async_copy(g_hbm.at[pl.ds(row0, ROWS_PER_W)],
                              g_s.at[pl.ds(row0, ROWS_PER_W)], sem1)
        cz = pltpu.async_copy(zeros_hbm, acc_s.at[pl.ds(row0, ROWS_PER_W)],
                              sem2)

        @pl.when(wid < w_split)
        def _():
            pltpu.async_copy(ei_hbm.at[pl.ds(wid * nit, nit)], idx_v,
                             sem0).wait()

        @pl.when(wid >= w_split)
        def _():
            pltpu.async_copy(tail_hbm.at[pl.ds((wid - w_split) * nit, nit)],
                             idx_v, sem0).wait()

        cg.wait()
        cz.wait()
        plsc.subcore_barrier()

        def issue(slot, c):
            pltpu.async_copy(g_s.at[idx_v.at[c, 0]], rows.at[slot],
                             sems[slot])

        def wait(slot):
            pltpu.make_async_copy(g_s.at[idx_v.at[0, 0]], rows.at[slot],
                                  sems[slot]).wait()

        for j in range(NBUF - 1):
            issue(j, j)

        def step(p, carry):
            c0 = p * NBUF
            for j in range(NBUF):
                c = c0 + j
                cn = c + NBUF - 1

                @pl.when(cn < nit)
                def _():
                    issue((j + NBUF - 1) % NBUF, cn)

                wait(j)
                pltpu.sync_copy(rows.at[j], acc_s.at[idx_v.at[c, 1]],
                                add=True)
            return carry

        lax.fori_loop(0, nit // NBUF, step, 0)
        plsc.subcore_barrier()
        pltpu.sync_copy(acc_s.at[pl.ds(row0, ROWS_PER_W)],
                        out_hbm.at[cid, pl.ds(row0, ROWS_PER_W)])

    return mp_k


def _dmat(degp_ref, ones_ref):
    deg = degp_ref[0:1, :] + degp_ref[1:2, :]
    dinv = jnp.where(deg > 0.0, lax.rsqrt(deg), 0.0)
    return lax.dot_general(dinv, ones_ref[...], (((0,), (0,)), ((), ())),
                           preferred_element_type=jnp.float32)


def _tc1_body(degp_ref, x_ref, w1_ref, ones_ref, g1_ref):
    n = x_ref.shape[0]
    dmat = _dmat(degp_ref, ones_ref)
    g1_ref[0:n] = jnp.dot(x_ref[...], w1_ref[...],
                          preferred_element_type=jnp.float32) * dmat[0:n]
    g1_ref[n:] = jnp.zeros((N_PAD - n, D_HID), jnp.float32)


def _tc2_body(sp_ref, degp_ref, ones_ref, b_ref, w2_ref, g2_ref):
    dmat = _dmat(degp_ref, ones_ref)
    s = sp_ref[0] + sp_ref[1]
    h = jnp.maximum(s * dmat + b_ref[...], 0.0)
    g2_ref[...] = jnp.dot(h, w2_ref[...],
                          preferred_element_type=jnp.float32) * dmat


def _tc3_body(sp_ref, degp_ref, ones_ref, b_ref, wfc_ref, bfc_ref, q_ref):
    dmat = _dmat(degp_ref, ones_ref)
    s = sp_ref[0] + sp_ref[1]
    h = jnp.maximum(s * dmat + b_ref[...], 0.0)
    q_ref[...] = jnp.dot(h, wfc_ref[...],
                         preferred_element_type=jnp.float32) + bfc_ref[...]


def kernel(x, edge_index, W1, b1, W2, b2, Wfc, bfc):
    n = x.shape[0]
    e = edge_index.shape[1]
    n_cls = Wfc.shape[1]
    e_tot = e + n
    grp = NW * CHUNK * NBUF
    e_pad = -(-e_tot // grp) * grp
    nit = e_pad // (NW * CHUNK)

    epw = nit * CHUNK
    w_split = e // epw
    ei32 = edge_index.astype(jnp.int32)
    ei_packed = jnp.stack(
        [ei32[0].reshape(-1, CHUNK), ei32[1].reshape(-1, CHUNK)],
        axis=1)
    loop = jnp.arange(n, dtype=jnp.int32)
    padv = jnp.full((e_pad - e_tot,), n, dtype=jnp.int32)
    tail_src = jnp.concatenate([ei32[0, w_split * epw:], loop, padv])
    tail_dst = jnp.concatenate([ei32[1, w_split * epw:], loop, padv])
    tail_packed = jnp.stack(
        [tail_src.reshape(-1, CHUNK), tail_dst.reshape(-1, CHUNK)],
        axis=1)

    zeros1d = jnp.zeros((ROWS_PER_W,), jnp.float32)
    ones1d = jnp.ones((CHUNK,), jnp.float32)
    zeros2d = jnp.zeros((ROWS_PER_W, D_HID), jnp.float32)
    ones_row = jnp.ones((1, D_HID), jnp.float32)

    deg_k = _make_deg_kernel(nit, w_split)
    mp_k = _make_mp_kernel(nit, w_split)

    degp = deg_k(ei_packed, tail_packed, zeros1d, ones1d)

    g1 = pl.pallas_call(
        _tc1_body,
        out_shape=jax.ShapeDtypeStruct((N_PAD, D_HID), jnp.float32),
    )(degp, x, W1, ones_row)

    s1p = mp_k(g1, ei_packed, tail_packed, zeros2d)

    g2 = pl.pallas_call(
        _tc2_body,
        out_shape=jax.ShapeDtypeStruct((N_PAD, D_HID), jnp.float32),
    )(s1p, degp, ones_row, b1.reshape(1, D_HID), W2)

    s2p = mp_k(g2, ei_packed, tail_packed, zeros2d)

    q = pl.pallas_call(
        _tc3_body,
        out_shape=jax.ShapeDtypeStruct((N_PAD, n_cls), jnp.float32),
    )(s2p, degp, ones_row, b2.reshape(1, D_HID), Wfc, bfc.reshape(1, n_cls))

    return q[:n]
